# baseline (speedup 1.0000x reference)
import functools

import jax
import jax.numpy as jnp
from jax import lax
from jax.experimental import pallas as pl
from jax.experimental.pallas import tpu as pltpu
from jax.experimental.pallas import tpu_sc as plsc

NC = 2
NS = 16
NW = NC * NS
LANES = 16

_CHUNK = 1024
_GB = 128

_PARAMS = pltpu.CompilerParams(
    needs_layout_passes=False, use_tc_tiling_on_sc=False)


def _rsqrt(x):
    xi = plsc.bitcast(x, jnp.int32)
    y = plsc.bitcast(jnp.int32(0x5F3759DF) - (xi >> 1), jnp.float32)
    for _ in range(3):
        y = y * (jnp.float32(1.5) - jnp.float32(0.5) * x * y * y)
    return y


def _full(v):
    return jnp.full((LANES,), v, jnp.int32)


def _build_table(pos_flat, batch_idx, lat_flat, n_batches):
    n = batch_idx.shape[0]
    nt = -(-n // (NW * 16)) * 16
    lat_words = lat_flat.shape[0]
    mesh = plsc.VectorSubcoreMesh(core_axis_name="c", subcore_axis_name="s")

    @functools.partial(
        pl.kernel,
        mesh=mesh,
        compiler_params=_PARAMS,
        out_type=[jax.ShapeDtypeStruct((n, 16), jnp.float32)],
        scratch_types=[
            pltpu.VMEM((nt * 3,), jnp.float32),
            pltpu.VMEM((nt,), jnp.int32),
            pltpu.VMEM((lat_words,), jnp.float32),
            pltpu.VMEM((nt, 16), jnp.float32),
        ],
    )
    def build(pos_hbm, b_hbm, lat_hbm, t_hbm, posb, bb, latb, tb):
        wid = lax.axis_index("s") * NC + lax.axis_index("c")
        base = jnp.minimum(wid * nt, n - nt)
        pltpu.sync_copy(pos_hbm.at[pl.ds(base * 3, nt * 3)], posb)
        pltpu.sync_copy(b_hbm.at[pl.ds(base, nt)], bb)
        pltpu.sync_copy(lat_hbm, latb)
        viota = lax.iota(jnp.int32, 16)

        def body(blk, carry):
            rows = blk * 16 + viota
            rows3 = rows * 3
            b = bb[pl.ds(blk * 16, 16)]
            b9 = jnp.clip(b, 0, n_batches - 1) * 9
            for k in range(3):
                p = plsc.load_gather(posb, [rows3 + k])
                plsc.store_scatter(tb, [rows, _full(k)], p)
            for mk in range(9):
                lv = plsc.load_gather(latb, [b9 + mk])
                plsc.store_scatter(tb, [rows, _full(3 + mk)], lv)
            return carry

        lax.fori_loop(0, nt // 16, body, 0)
        pltpu.sync_copy(tb, t_hbm.at[pl.ds(base, nt)])

    return build(pos_flat, batch_idx, lat_flat)


def _edge_kernel(t_tab, ej, ei, sh_flat):
    e = ej.shape[0]
    ept = e // NW
    n_chunks = -(-ept // _CHUNK)
    cb = _CHUNK // _GB
    mesh = plsc.VectorSubcoreMesh(core_axis_name="c", subcore_axis_name="s")

    @functools.partial(
        pl.kernel,
        mesh=mesh,
        compiler_params=_PARAMS,
        out_type=[
            jax.ShapeDtypeStruct((e,), jnp.float32),
            jax.ShapeDtypeStruct((e * 3,), jnp.float32),
            jax.ShapeDtypeStruct((e * 3,), jnp.float32),
        ],
        scratch_types=[
            pltpu.VMEM((_CHUNK,), jnp.int32),
            pltpu.VMEM((_CHUNK,), jnp.int32),
            pltpu.VMEM((_CHUNK * 3,), jnp.float32),
            pltpu.VMEM((cb, _GB, 16), jnp.float32),
            pltpu.VMEM((cb, _GB, 16), jnp.float32),
            pltpu.VMEM((_CHUNK,), jnp.float32),
            pltpu.VMEM((_CHUNK * 3,), jnp.float32),
            pltpu.VMEM((_CHUNK * 3,), jnp.float32),
            pltpu.SemaphoreType.DMA,
            pltpu.SemaphoreType.DMA,
        ],
    )
    def edges(t_hbm, ej_hbm, ei_hbm, sh_hbm,
              dist_hbm, vec_hbm, dir_hbm,
              jidx, iidx, shb, irows, jrows, distb, vecb, dirb,
              sem_i, sem_j):
        wid = lax.axis_index("s") * NC + lax.axis_index("c")
        tbase = wid * ept
        viota = lax.iota(jnp.int32, 16)

        def chunk_body(c, carry):
            g = tbase + jnp.minimum(c * _CHUNK, ept - _CHUNK)
            pltpu.sync_copy(ej_hbm.at[pl.ds(g, _CHUNK)], jidx)
            pltpu.sync_copy(ei_hbm.at[pl.ds(g, _CHUNK)], iidx)
            pltpu.sync_copy(sh_hbm.at[pl.ds(g * 3, _CHUNK * 3)], shb)
            copies = []
            for k in range(cb):
                copies.append(pltpu.async_copy(
                    t_hbm.at[iidx.at[pl.ds(k * _GB, _GB)]], irows.at[k],
                    sem_i))
            for k in range(cb):
                copies.append(pltpu.async_copy(
                    t_hbm.at[jidx.at[pl.ds(k * _GB, _GB)]], jrows.at[k],
                    sem_j))
            for cp in copies:
                cp.wait()

            def blk(bi, carry2):
                rows = bi * 16 + viota
                rows3 = rows * 3
                q = rows >> 7
                w = rows & 127
                s0 = plsc.load_gather(shb, [rows3])
                s1 = plsc.load_gather(shb, [rows3 + 1])
                s2 = plsc.load_gather(shb, [rows3 + 2])
                v = []
                for k in range(3):
                    pj = plsc.load_gather(jrows, [q, w, _full(k)])
                    pi = plsc.load_gather(irows, [q, w, _full(k)])
                    l0 = plsc.load_gather(irows, [q, w, _full(3 + k)])
                    l1 = plsc.load_gather(irows, [q, w, _full(6 + k)])
                    l2 = plsc.load_gather(irows, [q, w, _full(9 + k)])
                    v.append(pj - pi + s0 * l0 + s1 * l1 + s2 * l2)
                d2 = v[0] * v[0] + v[1] * v[1] + v[2] * v[2]
                y = _rsqrt(d2)
                distb[pl.ds(bi * 16, 16)] = d2 * y
                for k in range(3):
                    plsc.store_scatter(vecb, [rows3 + k], v[k])
                    plsc.store_scatter(dirb, [rows3 + k], v[k] * y)
                return carry2

            lax.fori_loop(0, _CHUNK // 16, blk, 0)
            pltpu.sync_copy(distb, dist_hbm.at[pl.ds(g, _CHUNK)])
            pltpu.sync_copy(vecb, vec_hbm.at[pl.ds(g * 3, _CHUNK * 3)])
            pltpu.sync_copy(dirb, dir_hbm.at[pl.ds(g * 3, _CHUNK * 3)])
            return carry

        lax.fori_loop(0, n_chunks, chunk_body, 0)

    return edges(t_tab, ej, ei, sh_flat)


def kernel(pos, edge_shift, lattice, edge_index, batch_idx):
    n = pos.shape[0]
    e = edge_shift.shape[0]
    b = lattice.shape[0]
    lat_flat = lattice.reshape(b * 9)
    pos_flat = pos.reshape(n * 3)

    nt = -(-n // (NW * 16)) * 16
    if n < nt or (n - nt) % 16 != 0:
        npad = nt * NW
        pos_flat = jnp.concatenate(
            [pos_flat, jnp.zeros(((npad - n) * 3,), pos.dtype)])
        batch_idx = jnp.concatenate(
            [batch_idx, jnp.zeros((npad - n,), batch_idx.dtype)])
    t_tab, = _build_table(pos_flat, batch_idx, lat_flat, b)

    ej = edge_index[0]
    ei = edge_index[1]
    sh_flat = edge_shift.reshape(e * 3)
    epad = e
    ept = e // NW
    if e % NW != 0 or ept % 16 != 0 or ept < _CHUNK:
        step = NW * _CHUNK
        epad = -(-e // step) * step
        pad = epad - e
        zi = jnp.zeros((pad,), jnp.int32)
        ej = jnp.concatenate([ej, zi])
        ei = jnp.concatenate([ei, zi])
        sh_flat = jnp.concatenate(
            [sh_flat, jnp.zeros((pad * 3,), edge_shift.dtype)])

    dist, vec_flat, dir_flat = _edge_kernel(t_tab, ej, ei, sh_flat)
    vec = vec_flat.reshape(epad, 3)
    dirn = dir_flat.reshape(epad, 3)
    if epad != e:
        dist, vec, dirn = dist[:e], vec[:e], dirn[:e]
    return dist, vec, dirn


# speedup vs baseline: 7.7125x; 7.7125x over previous
import functools

import jax
import jax.numpy as jnp
from jax import lax
from jax.experimental import pallas as pl
from jax.experimental.pallas import tpu as pltpu
from jax.experimental.pallas import tpu_sc as plsc

NC = 2
NS = 16
NW = NC * NS
LANES = 16

_CHUNK = 1024
_GB = 128

_PARAMS = pltpu.CompilerParams(
    needs_layout_passes=False, use_tc_tiling_on_sc=False)


def _rsqrt(x):
    xi = plsc.bitcast(x, jnp.int32)
    y = plsc.bitcast(jnp.int32(0x5F3759DF) - (xi >> 1), jnp.float32)
    for _ in range(3):
        y = y * (jnp.float32(1.5) - jnp.float32(0.5) * x * y * y)
    return y


def _full(v):
    return jnp.full((LANES,), v, jnp.int32)


def _build_table(pos_flat, batch_idx, lat_flat, n_batches):
    n = batch_idx.shape[0]
    nt = -(-n // (NW * 16)) * 16
    lat_words = lat_flat.shape[0]
    mesh = plsc.VectorSubcoreMesh(core_axis_name="c", subcore_axis_name="s")

    @functools.partial(
        pl.kernel,
        mesh=mesh,
        compiler_params=_PARAMS,
        out_type=[jax.ShapeDtypeStruct((n, 16), jnp.float32)],
        scratch_types=[
            pltpu.VMEM((nt * 3,), jnp.float32),
            pltpu.VMEM((nt,), jnp.int32),
            pltpu.VMEM((lat_words,), jnp.float32),
            pltpu.VMEM((nt, 16), jnp.float32),
        ],
    )
    def build(pos_hbm, b_hbm, lat_hbm, t_hbm, posb, bb, latb, tb):
        wid = lax.axis_index("s") * NC + lax.axis_index("c")
        base = jnp.minimum(wid * nt, n - nt)
        pltpu.sync_copy(pos_hbm.at[pl.ds(base * 3, nt * 3)], posb)
        pltpu.sync_copy(b_hbm.at[pl.ds(base, nt)], bb)
        pltpu.sync_copy(lat_hbm, latb)
        viota = lax.iota(jnp.int32, 16)

        def body(blk, carry):
            rows = blk * 16 + viota
            rows3 = rows * 3
            b = bb[pl.ds(blk * 16, 16)]
            b9 = jnp.clip(b, 0, n_batches - 1) * 9
            for k in range(3):
                p = plsc.load_gather(posb, [rows3 + k])
                plsc.store_scatter(tb, [rows, _full(k)], p)
            for mk in range(9):
                lv = plsc.load_gather(latb, [b9 + mk])
                plsc.store_scatter(tb, [rows, _full(3 + mk)], lv)
            return carry

        lax.fori_loop(0, nt // 16, body, 0)
        pltpu.sync_copy(tb, t_hbm.at[pl.ds(base, nt)])

    return build(pos_flat, batch_idx, lat_flat)


def _edge_kernel(t_tab, ej, ei, s0a, s1a, s2a):
    e = ej.shape[0]
    ept = e // NW
    n_chunks = -(-ept // _CHUNK)
    cb = _CHUNK // _GB
    mesh = plsc.VectorSubcoreMesh(core_axis_name="c", subcore_axis_name="s")
    vout = jax.ShapeDtypeStruct((e,), jnp.float32)

    @functools.partial(
        pl.kernel,
        mesh=mesh,
        compiler_params=_PARAMS,
        out_type=[vout] * 7,
        scratch_types=[
            pltpu.VMEM((_CHUNK,), jnp.int32),
            pltpu.VMEM((_CHUNK,), jnp.int32),
            pltpu.VMEM((_CHUNK,), jnp.float32),
            pltpu.VMEM((_CHUNK,), jnp.float32),
            pltpu.VMEM((_CHUNK,), jnp.float32),
            pltpu.VMEM((cb, _GB, 16), jnp.float32),
            pltpu.VMEM((cb, _GB, 16), jnp.float32),
            pltpu.VMEM((7, _CHUNK), jnp.float32),
            pltpu.SemaphoreType.DMA,
            pltpu.SemaphoreType.DMA,
        ],
    )
    def edges(t_hbm, ej_hbm, ei_hbm, s0_hbm, s1_hbm, s2_hbm,
              dist_hbm, vx_hbm, vy_hbm, vz_hbm, dx_hbm, dy_hbm, dz_hbm,
              jidx, iidx, s0b, s1b, s2b, irows, jrows, outb,
              sem_i, sem_j):
        wid = lax.axis_index("s") * NC + lax.axis_index("c")
        tbase = wid * ept
        viota = lax.iota(jnp.int32, 16)
        out_hbms = (dist_hbm, vx_hbm, vy_hbm, vz_hbm, dx_hbm, dy_hbm, dz_hbm)

        def chunk_body(c, carry):
            g = tbase + jnp.minimum(c * _CHUNK, ept - _CHUNK)
            pltpu.sync_copy(ej_hbm.at[pl.ds(g, _CHUNK)], jidx)
            pltpu.sync_copy(ei_hbm.at[pl.ds(g, _CHUNK)], iidx)
            pltpu.sync_copy(s0_hbm.at[pl.ds(g, _CHUNK)], s0b)
            pltpu.sync_copy(s1_hbm.at[pl.ds(g, _CHUNK)], s1b)
            pltpu.sync_copy(s2_hbm.at[pl.ds(g, _CHUNK)], s2b)
            copies = []
            for k in range(cb):
                copies.append(pltpu.async_copy(
                    t_hbm.at[iidx.at[pl.ds(k * _GB, _GB)]], irows.at[k],
                    sem_i))
            for k in range(cb):
                copies.append(pltpu.async_copy(
                    t_hbm.at[jidx.at[pl.ds(k * _GB, _GB)]], jrows.at[k],
                    sem_j))
            for cp in copies:
                cp.wait()

            def blk(bi, carry2):
                o = bi * 16
                rows = o + viota
                q = rows >> 7
                w = rows & 127
                s0 = s0b[pl.ds(o, 16)]
                s1 = s1b[pl.ds(o, 16)]
                s2 = s2b[pl.ds(o, 16)]
                v = []
                for k in range(3):
                    pj = plsc.load_gather(jrows, [q, w, _full(k)])
                    pi = plsc.load_gather(irows, [q, w, _full(k)])
                    l0 = plsc.load_gather(irows, [q, w, _full(3 + k)])
                    l1 = plsc.load_gather(irows, [q, w, _full(6 + k)])
                    l2 = plsc.load_gather(irows, [q, w, _full(9 + k)])
                    v.append(pj - pi + s0 * l0 + s1 * l1 + s2 * l2)
                d2 = v[0] * v[0] + v[1] * v[1] + v[2] * v[2]
                y = _rsqrt(d2)
                outb[0, pl.ds(o, 16)] = d2 * y
                for k in range(3):
                    outb[1 + k, pl.ds(o, 16)] = v[k]
                    outb[4 + k, pl.ds(o, 16)] = v[k] * y
                return carry2

            lax.fori_loop(0, _CHUNK // 16, blk, 0)
            for t, hbm in enumerate(out_hbms):
                pltpu.sync_copy(outb.at[t], hbm.at[pl.ds(g, _CHUNK)])
            return carry

        lax.fori_loop(0, n_chunks, chunk_body, 0)

    return edges(t_tab, ej, ei, s0a, s1a, s2a)


def kernel(pos, edge_shift, lattice, edge_index, batch_idx):
    n = pos.shape[0]
    e = edge_shift.shape[0]
    b = lattice.shape[0]
    lat_flat = lattice.reshape(b * 9)
    pos_flat = pos.reshape(n * 3)

    nt = -(-n // (NW * 16)) * 16
    if n < nt or (n - nt) % 16 != 0:
        npad = nt * NW
        pos_flat = jnp.concatenate(
            [pos_flat, jnp.zeros(((npad - n) * 3,), pos.dtype)])
        batch_idx = jnp.concatenate(
            [batch_idx, jnp.zeros((npad - n,), batch_idx.dtype)])
    t_tab, = _build_table(pos_flat, batch_idx, lat_flat, b)

    ej = edge_index[0]
    ei = edge_index[1]
    s0a = edge_shift[:, 0]
    s1a = edge_shift[:, 1]
    s2a = edge_shift[:, 2]
    epad = e
    ept = e // NW
    if e % NW != 0 or ept % 16 != 0 or ept < _CHUNK:
        step = NW * _CHUNK
        epad = -(-e // step) * step
        pad = epad - e
        zi = jnp.zeros((pad,), jnp.int32)
        zf = jnp.zeros((pad,), edge_shift.dtype)
        ej = jnp.concatenate([ej, zi])
        ei = jnp.concatenate([ei, zi])
        s0a = jnp.concatenate([s0a, zf])
        s1a = jnp.concatenate([s1a, zf])
        s2a = jnp.concatenate([s2a, zf])

    dist, vx, vy, vz, dx, dy, dz = _edge_kernel(t_tab, ej, ei, s0a, s1a, s2a)
    vec = jnp.stack([vx, vy, vz], axis=1)
    dirn = jnp.stack([dx, dy, dz], axis=1)
    if epad != e:
        dist, vec, dirn = dist[:e], vec[:e], dirn[:e]
    return dist, vec, dirn


# speedup vs baseline: 9.8580x; 1.2782x over previous
import functools

import jax
import jax.numpy as jnp
from jax import lax
from jax.experimental import pallas as pl
from jax.experimental.pallas import tpu as pltpu
from jax.experimental.pallas import tpu_sc as plsc

NC = 2
NS = 16
NW = NC * NS
LANES = 16

_CHUNK = 1024
_GB = 128

_PARAMS = pltpu.CompilerParams(
    needs_layout_passes=False, use_tc_tiling_on_sc=False)


def _rsqrt(x):
    xi = plsc.bitcast(x, jnp.int32)
    y = plsc.bitcast(jnp.int32(0x5F3759DF) - (xi >> 1), jnp.float32)
    for _ in range(3):
        y = y * (jnp.float32(1.5) - jnp.float32(0.5) * x * y * y)
    return y


def _full(v):
    return jnp.full((LANES,), v, jnp.int32)


def _build_table(pos_flat, batch_idx, lat_flat, n_batches):
    n = batch_idx.shape[0]
    nt = -(-n // (NW * 16)) * 16
    lat_words = lat_flat.shape[0]
    mesh = plsc.VectorSubcoreMesh(core_axis_name="c", subcore_axis_name="s")

    @functools.partial(
        pl.kernel,
        mesh=mesh,
        compiler_params=_PARAMS,
        out_type=[jax.ShapeDtypeStruct((n, 16), jnp.float32)],
        scratch_types=[
            pltpu.VMEM((nt * 3,), jnp.float32),
            pltpu.VMEM((nt,), jnp.int32),
            pltpu.VMEM((lat_words,), jnp.float32),
            pltpu.VMEM((nt, 16), jnp.float32),
        ],
    )
    def build(pos_hbm, b_hbm, lat_hbm, t_hbm, posb, bb, latb, tb):
        wid = lax.axis_index("s") * NC + lax.axis_index("c")
        base = jnp.minimum(wid * nt, n - nt)
        pltpu.sync_copy(pos_hbm.at[pl.ds(base * 3, nt * 3)], posb)
        pltpu.sync_copy(b_hbm.at[pl.ds(base, nt)], bb)
        pltpu.sync_copy(lat_hbm, latb)
        viota = lax.iota(jnp.int32, 16)

        def body(blk, carry):
            rows = blk * 16 + viota
            rows3 = rows * 3
            b = bb[pl.ds(blk * 16, 16)]
            b9 = jnp.clip(b, 0, n_batches - 1) * 9
            for k in range(3):
                p = plsc.load_gather(posb, [rows3 + k])
                plsc.store_scatter(tb, [rows, _full(k)], p)
            for mk in range(9):
                lv = plsc.load_gather(latb, [b9 + mk])
                plsc.store_scatter(tb, [rows, _full(3 + mk)], lv)
            return carry

        lax.fori_loop(0, nt // 16, body, 0)
        pltpu.sync_copy(tb, t_hbm.at[pl.ds(base, nt)])

    return build(pos_flat, batch_idx, lat_flat)


def _edge_kernel(t_tab, ej, ei, s0a, s1a, s2a):
    e = ej.shape[0]
    ept = e // NW
    n_chunks = -(-ept // _CHUNK)
    n_pairs = -(-n_chunks // 2)
    cb = _CHUNK // _GB
    mesh = plsc.VectorSubcoreMesh(core_axis_name="c", subcore_axis_name="s")
    vout = jax.ShapeDtypeStruct((e,), jnp.float32)
    buf_types = [
        pltpu.VMEM((_CHUNK,), jnp.int32),
        pltpu.VMEM((_CHUNK,), jnp.int32),
        pltpu.VMEM((_CHUNK,), jnp.float32),
        pltpu.VMEM((_CHUNK,), jnp.float32),
        pltpu.VMEM((_CHUNK,), jnp.float32),
        pltpu.VMEM((cb, _GB, 16), jnp.float32),
        pltpu.VMEM((cb, _GB, 16), jnp.float32),
        pltpu.VMEM((7, _CHUNK), jnp.float32),
    ]

    @functools.partial(
        pl.kernel,
        mesh=mesh,
        compiler_params=_PARAMS,
        out_type=[vout] * 7,
        scratch_types=buf_types + buf_types + [
            pltpu.SemaphoreType.DMA,
            pltpu.SemaphoreType.DMA,
            pltpu.SemaphoreType.DMA,
            pltpu.SemaphoreType.DMA,
        ],
    )
    def edges(t_hbm, ej_hbm, ei_hbm, s0_hbm, s1_hbm, s2_hbm,
              dist_hbm, vx_hbm, vy_hbm, vz_hbm, dx_hbm, dy_hbm, dz_hbm,
              *refs):
        bufs = (refs[0:8], refs[8:16])
        sg = (refs[16], refs[17])
        sout = (refs[18], refs[19])
        wid = lax.axis_index("s") * NC + lax.axis_index("c")
        tbase = wid * ept
        viota = lax.iota(jnp.int32, 16)
        out_hbms = (dist_hbm, vx_hbm, vy_hbm, vz_hbm, dx_hbm, dy_hbm, dz_hbm)
        in_hbms = (ej_hbm, ei_hbm, s0_hbm, s1_hbm, s2_hbm)

        def chunk_start(c):
            return tbase + jnp.minimum(c * _CHUNK, ept - _CHUNK)

        def sync_in(c, b):
            g = chunk_start(c)
            for src_h, dst in zip(in_hbms, bufs[b][:5]):
                pltpu.sync_copy(src_h.at[pl.ds(g, _CHUNK)], dst)

        def fire_gathers(b):
            jidx, iidx = bufs[b][0], bufs[b][1]
            irows, jrows = bufs[b][5], bufs[b][6]
            cps = []
            for k in range(cb):
                cps.append(pltpu.async_copy(
                    t_hbm.at[iidx.at[pl.ds(k * _GB, _GB)]], irows.at[k],
                    sg[b]))
                cps.append(pltpu.async_copy(
                    t_hbm.at[jidx.at[pl.ds(k * _GB, _GB)]], jrows.at[k],
                    sg[b]))
            return cps

        def wait_gathers(b):
            jidx, iidx = bufs[b][0], bufs[b][1]
            irows, jrows = bufs[b][5], bufs[b][6]
            for k in range(cb):
                pltpu.make_async_copy(
                    t_hbm.at[iidx.at[pl.ds(k * _GB, _GB)]], irows.at[k],
                    sg[b]).wait()
                pltpu.make_async_copy(
                    t_hbm.at[jidx.at[pl.ds(k * _GB, _GB)]], jrows.at[k],
                    sg[b]).wait()

        def wait_outs(b):
            outb = bufs[b][7]
            for t, hbm in enumerate(out_hbms):
                pltpu.make_async_copy(
                    outb.at[t], hbm.at[pl.ds(tbase, _CHUNK)], sout[b]).wait()

        def compute(b):
            s0b, s1b, s2b = bufs[b][2], bufs[b][3], bufs[b][4]
            irows, jrows = bufs[b][5], bufs[b][6]
            outb = bufs[b][7]

            def blk(bi, carry2):
                o = bi * 16
                rows = o + viota
                q = rows >> 7
                w = rows & 127
                s0 = s0b[pl.ds(o, 16)]
                s1 = s1b[pl.ds(o, 16)]
                s2 = s2b[pl.ds(o, 16)]
                v = []
                for k in range(3):
                    pj = plsc.load_gather(jrows, [q, w, _full(k)])
                    pi = plsc.load_gather(irows, [q, w, _full(k)])
                    l0 = plsc.load_gather(irows, [q, w, _full(3 + k)])
                    l1 = plsc.load_gather(irows, [q, w, _full(6 + k)])
                    l2 = plsc.load_gather(irows, [q, w, _full(9 + k)])
                    v.append(pj - pi + s0 * l0 + s1 * l1 + s2 * l2)
                d2 = v[0] * v[0] + v[1] * v[1] + v[2] * v[2]
                y = _rsqrt(d2)
                outb[0, pl.ds(o, 16)] = d2 * y
                for k in range(3):
                    outb[1 + k, pl.ds(o, 16)] = v[k]
                    outb[4 + k, pl.ds(o, 16)] = v[k] * y
                return carry2

            lax.fori_loop(0, _CHUNK // 16, blk, 0)

        def fire_outs(c, b):
            g = chunk_start(c)
            outb = bufs[b][7]
            for t, hbm in enumerate(out_hbms):
                pltpu.async_copy(outb.at[t], hbm.at[pl.ds(g, _CHUNK)],
                                 sout[b])

        sync_in(0, 0)
        fire_gathers(0)

        def pair_body(m, carry):
            c0 = 2 * m
            c1 = c0 + 1
            sync_in(c1, 1)
            fire_gathers(1)
            wait_gathers(0)

            @pl.when(m > 0)
            def _():
                wait_outs(0)

            compute(0)
            fire_outs(c0, 0)

            @pl.when(m + 1 < n_pairs)
            def _():
                sync_in(c0 + 2, 0)
                fire_gathers(0)

            wait_gathers(1)

            @pl.when(m > 0)
            def _():
                wait_outs(1)

            compute(1)
            fire_outs(c1, 1)
            return carry

        lax.fori_loop(0, n_pairs, pair_body, 0)
        wait_outs(0)
        wait_outs(1)

    return edges(t_tab, ej, ei, s0a, s1a, s2a)


def kernel(pos, edge_shift, lattice, edge_index, batch_idx):
    n = pos.shape[0]
    e = edge_shift.shape[0]
    b = lattice.shape[0]
    lat_flat = lattice.reshape(b * 9)
    pos_flat = pos.reshape(n * 3)

    nt = -(-n // (NW * 16)) * 16
    if n < nt or (n - nt) % 16 != 0:
        npad = nt * NW
        pos_flat = jnp.concatenate(
            [pos_flat, jnp.zeros(((npad - n) * 3,), pos.dtype)])
        batch_idx = jnp.concatenate(
            [batch_idx, jnp.zeros((npad - n,), batch_idx.dtype)])
    t_tab, = _build_table(pos_flat, batch_idx, lat_flat, b)

    ej = edge_index[0]
    ei = edge_index[1]
    s0a = edge_shift[:, 0]
    s1a = edge_shift[:, 1]
    s2a = edge_shift[:, 2]
    epad = e
    ept = e // NW
    if e % NW != 0 or ept % 16 != 0 or ept < _CHUNK:
        step = NW * _CHUNK
        epad = -(-e // step) * step
        pad = epad - e
        zi = jnp.zeros((pad,), jnp.int32)
        zf = jnp.zeros((pad,), edge_shift.dtype)
        ej = jnp.concatenate([ej, zi])
        ei = jnp.concatenate([ei, zi])
        s0a = jnp.concatenate([s0a, zf])
        s1a = jnp.concatenate([s1a, zf])
        s2a = jnp.concatenate([s2a, zf])

    dist, vx, vy, vz, dx, dy, dz = _edge_kernel(t_tab, ej, ei, s0a, s1a, s2a)
    vec = jnp.stack([vx, vy, vz], axis=1)
    dirn = jnp.stack([dx, dy, dz], axis=1)
    if epad != e:
        dist, vec, dirn = dist[:e], vec[:e], dirn[:e]
    return dist, vec, dirn
